# baseline (speedup 1.0000x reference)
import jax
import jax.numpy as jnp
import numpy as np
from jax.experimental import pallas as pl
from jax.experimental.pallas import tpu as pltpu

DIM_M = 64
D_MAX = 65
OMEGA = 0.1
K_COEF = 1.0
SCALE = DIM_M ** 0.5

N = 10000
F = 128
ROW_BLK = 200
N_ROW = N // ROW_BLK
IDX_N = 1000


def _make_pe():
    pos = np.arange(D_MAX)[:, None].astype(np.float64)
    i = np.arange(DIM_M)[None, :].astype(np.float64)
    pe = pos / np.power(10000.0, (i - i % 2) / DIM_M)
    pe[:, 0::2] = np.sin(pe[:, 0::2])
    pe[:, 1::2] = np.cos(pe[:, 1::2])
    return jnp.asarray(pe, jnp.float32)


def _lrelu(v):
    return jnp.where(v >= 0, v, 0.01 * v)


def _main_kernel(adj_ref, x_ref, deg_ref, degfull_ref, idx_ref, pe_ref,
                 wl_ref, bl_ref, wg_ref, wb_ref, bg_ref, bb_ref,
                 wadd_ref, wrev_ref, out_ref, sums_ref):
    r = pl.program_id(0)

    adj_blk = adj_ref[...]
    p = jnp.dot(adj_blk, x_ref[...], preferred_element_type=jnp.float32)
    rowsum = jnp.sum(adj_blk, axis=1, keepdims=True)
    agg = SCALE * (jax.lax.dot_general(
        p, wl_ref[...], (((1,), (1,)), ((), ())),
        preferred_element_type=jnp.float32) + rowsum * bl_ref[...])

    x_row = x_ref[pl.ds(r * ROW_BLK, ROW_BLK), :]
    h_row = SCALE * (jax.lax.dot_general(
        x_row, wl_ref[...], (((1,), (1,)), ((), ())),
        preferred_element_type=jnp.float32) + bl_ref[...])

    deg_i = deg_ref[...]
    deg_f = deg_i.astype(jnp.float32)
    is_zero = deg_f == 0.0
    i_feat = jnp.where(is_zero, 0.0, agg / jnp.where(is_zero, 1.0, deg_f))

    oh = (jax.lax.broadcasted_iota(jnp.int32, (ROW_BLK, D_MAX), 1)
          == deg_i).astype(jnp.float32)
    m_dv = jnp.dot(oh, pe_ref[...], preferred_element_type=jnp.float32)
    gamma = _lrelu(jnp.dot(m_dv, wg_ref[...],
                           preferred_element_type=jnp.float32) + bg_ref[...])
    beta = _lrelu(jnp.dot(m_dv, wb_ref[...],
                          preferred_element_type=jnp.float32) + bb_ref[...])

    g1 = gamma + 1.0
    b_add = g1 * jax.lax.dot_general(
        i_feat, wadd_ref[...], (((1,), (1,)), ((), ())),
        preferred_element_type=jnp.float32) + beta
    b_rev = g1 * jax.lax.dot_general(
        i_feat, wrev_ref[...], (((1,), (1,)), ((), ())),
        preferred_element_type=jnp.float32) + beta

    mean_deg = jnp.sum(degfull_ref[...].astype(jnp.float32)) / np.float32(N)
    r_mask = (deg_f < mean_deg * K_COEF).astype(jnp.float32)

    bias = OMEGA * (r_mask * b_add - (1.0 - r_mask) * b_rev)
    out_ref[...] = _lrelu((agg + h_row + bias) / (deg_f + 1.0))

    nrm = lambda v: jnp.sqrt(jnp.sum(v * v, axis=1, keepdims=True))
    n_bsel = r_mask * nrm(b_add) + (1.0 - r_mask) * nrm(b_rev)
    row_ids = (r * ROW_BLK
               + jax.lax.broadcasted_iota(jnp.int32, (ROW_BLK, 1), 0))
    cnt = jnp.sum((row_ids == idx_ref[...]).astype(jnp.float32),
                  axis=1, keepdims=True)
    part = jnp.concatenate([
        jnp.sum(cnt * n_bsel).reshape(1, 1),
        jnp.sum(cnt * nrm(gamma)).reshape(1, 1),
        jnp.sum(cnt * nrm(beta)).reshape(1, 1)], axis=1)

    @pl.when(r == 0)
    def _():
        sums_ref[...] = part

    @pl.when(r != 0)
    def _():
        sums_ref[...] += part


def kernel(x, adj, degree, idx, edge, Wl, bl, W_gamma, W_beta, b_gamma,
           b_beta, W_add, W_rev):
    pe = _make_pe()
    bl2 = bl.reshape(1, F)
    idx2 = idx.reshape(1, IDX_N)

    out, sums = pl.pallas_call(
        _main_kernel,
        grid=(N_ROW,),
        in_specs=[
            pl.BlockSpec((ROW_BLK, N), lambda r: (r, 0)),
            pl.BlockSpec((N, F), lambda r: (0, 0)),
            pl.BlockSpec((ROW_BLK, 1), lambda r: (r, 0)),
            pl.BlockSpec((N, 1), lambda r: (0, 0)),
            pl.BlockSpec((1, IDX_N), lambda r: (0, 0)),
            pl.BlockSpec((D_MAX, DIM_M), lambda r: (0, 0)),
            pl.BlockSpec((F, F), lambda r: (0, 0)),
            pl.BlockSpec((1, F), lambda r: (0, 0)),
            pl.BlockSpec((DIM_M, F), lambda r: (0, 0)),
            pl.BlockSpec((DIM_M, F), lambda r: (0, 0)),
            pl.BlockSpec((1, F), lambda r: (0, 0)),
            pl.BlockSpec((1, F), lambda r: (0, 0)),
            pl.BlockSpec((F, F), lambda r: (0, 0)),
            pl.BlockSpec((F, F), lambda r: (0, 0)),
        ],
        out_specs=[
            pl.BlockSpec((ROW_BLK, F), lambda r: (r, 0)),
            pl.BlockSpec((1, 3), lambda r: (0, 0)),
        ],
        out_shape=[
            jax.ShapeDtypeStruct((N, F), jnp.float32),
            jax.ShapeDtypeStruct((1, 3), jnp.float32),
        ],
    )(adj, x, degree, degree, idx2, pe, Wl, bl2, W_gamma, W_beta,
      b_gamma, b_beta, W_add, W_rev)

    inv = np.float32(1.0 / IDX_N)
    l_b = sums[0, 0] * inv
    l_film = (sums[0, 1] + sums[0, 2]) * inv
    return (out, l_b, l_film)


# speedup vs baseline: 1.1151x; 1.1151x over previous
import jax
import jax.numpy as jnp
import numpy as np
from jax.experimental import pallas as pl
from jax.experimental.pallas import tpu as pltpu

DIM_M = 64
D_MAX = 65
OMEGA = 0.1
K_COEF = 1.0
SCALE = DIM_M ** 0.5

N = 10000
F = 128
ROW_BLK = 400
N_ROW = N // ROW_BLK
IDX_N = 1000


def _make_pe():
    pos = np.arange(D_MAX)[:, None].astype(np.float64)
    i = np.arange(DIM_M)[None, :].astype(np.float64)
    pe = pos / np.power(10000.0, (i - i % 2) / DIM_M)
    pe[:, 0::2] = np.sin(pe[:, 0::2])
    pe[:, 1::2] = np.cos(pe[:, 1::2])
    return jnp.asarray(pe, jnp.float32)


def _lrelu(v):
    return jnp.where(v >= 0, v, 0.01 * v)


def _main_kernel(adj_ref, x_ref, deg_ref, degfull_ref, idx_ref, pe_ref,
                 wl_ref, bl_ref, wg_ref, wb_ref, bg_ref, bb_ref,
                 wadd_ref, wrev_ref, out_ref, sums_ref):
    r = pl.program_id(0)

    adj_blk = adj_ref[...]
    p = jnp.dot(adj_blk, x_ref[...], preferred_element_type=jnp.float32)
    rowsum = jnp.sum(adj_blk, axis=1, keepdims=True)
    agg = SCALE * (jax.lax.dot_general(
        p, wl_ref[...], (((1,), (1,)), ((), ())),
        preferred_element_type=jnp.float32) + rowsum * bl_ref[...])

    x_row = x_ref[pl.ds(r * ROW_BLK, ROW_BLK), :]
    h_row = SCALE * (jax.lax.dot_general(
        x_row, wl_ref[...], (((1,), (1,)), ((), ())),
        preferred_element_type=jnp.float32) + bl_ref[...])

    deg_i = deg_ref[...]
    deg_f = deg_i.astype(jnp.float32)
    is_zero = deg_f == 0.0
    i_feat = jnp.where(is_zero, 0.0, agg / jnp.where(is_zero, 1.0, deg_f))

    oh = (jax.lax.broadcasted_iota(jnp.int32, (ROW_BLK, D_MAX), 1)
          == deg_i).astype(jnp.float32)
    m_dv = jnp.dot(oh, pe_ref[...], preferred_element_type=jnp.float32)
    gamma = _lrelu(jnp.dot(m_dv, wg_ref[...],
                           preferred_element_type=jnp.float32) + bg_ref[...])
    beta = _lrelu(jnp.dot(m_dv, wb_ref[...],
                          preferred_element_type=jnp.float32) + bb_ref[...])

    g1 = gamma + 1.0
    b_add = g1 * jax.lax.dot_general(
        i_feat, wadd_ref[...], (((1,), (1,)), ((), ())),
        preferred_element_type=jnp.float32) + beta
    b_rev = g1 * jax.lax.dot_general(
        i_feat, wrev_ref[...], (((1,), (1,)), ((), ())),
        preferred_element_type=jnp.float32) + beta

    mean_deg = jnp.sum(degfull_ref[...].astype(jnp.float32)) / np.float32(N)
    r_mask = (deg_f < mean_deg * K_COEF).astype(jnp.float32)

    bias = OMEGA * (r_mask * b_add - (1.0 - r_mask) * b_rev)
    out_ref[...] = _lrelu((agg + h_row + bias) / (deg_f + 1.0))

    nrm = lambda v: jnp.sqrt(jnp.sum(v * v, axis=1, keepdims=True))
    n_bsel = r_mask * nrm(b_add) + (1.0 - r_mask) * nrm(b_rev)
    row_ids = (r * ROW_BLK
               + jax.lax.broadcasted_iota(jnp.int32, (ROW_BLK, 1), 0))
    cnt = jnp.sum((row_ids == idx_ref[...]).astype(jnp.float32),
                  axis=1, keepdims=True)
    part = jnp.concatenate([
        jnp.sum(cnt * n_bsel).reshape(1, 1),
        jnp.sum(cnt * nrm(gamma)).reshape(1, 1),
        jnp.sum(cnt * nrm(beta)).reshape(1, 1)], axis=1)

    @pl.when(r == 0)
    def _():
        sums_ref[...] = part

    @pl.when(r != 0)
    def _():
        sums_ref[...] += part


def kernel(x, adj, degree, idx, edge, Wl, bl, W_gamma, W_beta, b_gamma,
           b_beta, W_add, W_rev):
    pe = _make_pe()
    bl2 = bl.reshape(1, F)
    idx2 = idx.reshape(1, IDX_N)

    out, sums = pl.pallas_call(
        _main_kernel,
        grid=(N_ROW,),
        in_specs=[
            pl.BlockSpec((ROW_BLK, N), lambda r: (r, 0)),
            pl.BlockSpec((N, F), lambda r: (0, 0)),
            pl.BlockSpec((ROW_BLK, 1), lambda r: (r, 0)),
            pl.BlockSpec((N, 1), lambda r: (0, 0)),
            pl.BlockSpec((1, IDX_N), lambda r: (0, 0)),
            pl.BlockSpec((D_MAX, DIM_M), lambda r: (0, 0)),
            pl.BlockSpec((F, F), lambda r: (0, 0)),
            pl.BlockSpec((1, F), lambda r: (0, 0)),
            pl.BlockSpec((DIM_M, F), lambda r: (0, 0)),
            pl.BlockSpec((DIM_M, F), lambda r: (0, 0)),
            pl.BlockSpec((1, F), lambda r: (0, 0)),
            pl.BlockSpec((1, F), lambda r: (0, 0)),
            pl.BlockSpec((F, F), lambda r: (0, 0)),
            pl.BlockSpec((F, F), lambda r: (0, 0)),
        ],
        out_specs=[
            pl.BlockSpec((ROW_BLK, F), lambda r: (r, 0)),
            pl.BlockSpec((1, 3), lambda r: (0, 0)),
        ],
        out_shape=[
            jax.ShapeDtypeStruct((N, F), jnp.float32),
            jax.ShapeDtypeStruct((1, 3), jnp.float32),
        ],
    )(adj, x, degree, degree, idx2, pe, Wl, bl2, W_gamma, W_beta,
      b_gamma, b_beta, W_add, W_rev)

    inv = np.float32(1.0 / IDX_N)
    l_b = sums[0, 0] * inv
    l_film = (sums[0, 1] + sums[0, 2]) * inv
    return (out, l_b, l_film)


# speedup vs baseline: 1.1852x; 1.0628x over previous
import jax
import jax.numpy as jnp
import numpy as np
from jax.experimental import pallas as pl
from jax.experimental.pallas import tpu as pltpu

DIM_M = 64
D_MAX = 65
OMEGA = 0.1
K_COEF = 1.0
SCALE = DIM_M ** 0.5

N = 10000
F = 128
ROW_BLK = 400
N_ROW = N // ROW_BLK
IDX_N = 1000


def _make_pe():
    pos = np.arange(D_MAX)[:, None].astype(np.float64)
    i = np.arange(DIM_M)[None, :].astype(np.float64)
    pe = pos / np.power(10000.0, (i - i % 2) / DIM_M)
    pe[:, 0::2] = np.sin(pe[:, 0::2])
    pe[:, 1::2] = np.cos(pe[:, 1::2])
    return jnp.asarray(pe, jnp.float32)


def _lrelu(v):
    return jnp.where(v >= 0, v, 0.01 * v)


def _main_kernel(adj_ref, x_ref, deg_ref, degfull_ref, idx_ref, pe_ref,
                 wl_ref, bl_ref, wg_ref, wb_ref, bg_ref, bb_ref,
                 wadd_ref, wrev_ref, out_ref, sums_ref, h_ref, mean_ref):
    r = pl.program_id(0)

    @pl.when(r == 0)
    def _():
        h_ref[...] = SCALE * (jax.lax.dot_general(
            x_ref[...], wl_ref[...], (((1,), (1,)), ((), ())),
            preferred_element_type=jnp.float32) + bl_ref[...])
        mean_ref[0] = (jnp.sum(degfull_ref[...].astype(jnp.float32))
                       / np.float32(N))

    agg = jnp.dot(adj_ref[...], h_ref[...], preferred_element_type=jnp.float32)
    h_row = h_ref[pl.ds(r * ROW_BLK, ROW_BLK), :]

    deg_i = deg_ref[...]
    deg_f = deg_i.astype(jnp.float32)
    is_zero = deg_f == 0.0
    i_feat = jnp.where(is_zero, 0.0, agg / jnp.where(is_zero, 1.0, deg_f))

    oh = (jax.lax.broadcasted_iota(jnp.int32, (ROW_BLK, D_MAX), 1)
          == deg_i).astype(jnp.float32)
    m_dv = jnp.dot(oh, pe_ref[...], preferred_element_type=jnp.float32)
    gamma = _lrelu(jnp.dot(m_dv, wg_ref[...],
                           preferred_element_type=jnp.float32) + bg_ref[...])
    beta = _lrelu(jnp.dot(m_dv, wb_ref[...],
                          preferred_element_type=jnp.float32) + bb_ref[...])

    g1 = gamma + 1.0
    b_add = g1 * jax.lax.dot_general(
        i_feat, wadd_ref[...], (((1,), (1,)), ((), ())),
        preferred_element_type=jnp.float32) + beta
    b_rev = g1 * jax.lax.dot_general(
        i_feat, wrev_ref[...], (((1,), (1,)), ((), ())),
        preferred_element_type=jnp.float32) + beta

    r_mask = (deg_f < mean_ref[0] * K_COEF).astype(jnp.float32)

    bias = OMEGA * (r_mask * b_add - (1.0 - r_mask) * b_rev)
    out_ref[...] = _lrelu((agg + h_row + bias) / (deg_f + 1.0))

    nrm = lambda v: jnp.sqrt(jnp.sum(v * v, axis=1, keepdims=True))
    n_bsel = r_mask * nrm(b_add) + (1.0 - r_mask) * nrm(b_rev)
    row_ids = (r * ROW_BLK
               + jax.lax.broadcasted_iota(jnp.int32, (ROW_BLK, 1), 0))
    cnt = jnp.sum((row_ids == idx_ref[...]).astype(jnp.float32),
                  axis=1, keepdims=True)
    part = jnp.concatenate([
        jnp.sum(cnt * n_bsel).reshape(1, 1),
        jnp.sum(cnt * nrm(gamma)).reshape(1, 1),
        jnp.sum(cnt * nrm(beta)).reshape(1, 1)], axis=1)

    @pl.when(r == 0)
    def _():
        sums_ref[...] = part

    @pl.when(r != 0)
    def _():
        sums_ref[...] += part


def kernel(x, adj, degree, idx, edge, Wl, bl, W_gamma, W_beta, b_gamma,
           b_beta, W_add, W_rev):
    pe = _make_pe()
    bl2 = bl.reshape(1, F)
    idx2 = idx.reshape(1, IDX_N)

    out, sums = pl.pallas_call(
        _main_kernel,
        grid=(N_ROW,),
        in_specs=[
            pl.BlockSpec((ROW_BLK, N), lambda r: (r, 0)),
            pl.BlockSpec((N, F), lambda r: (0, 0)),
            pl.BlockSpec((ROW_BLK, 1), lambda r: (r, 0)),
            pl.BlockSpec((N, 1), lambda r: (0, 0)),
            pl.BlockSpec((1, IDX_N), lambda r: (0, 0)),
            pl.BlockSpec((D_MAX, DIM_M), lambda r: (0, 0)),
            pl.BlockSpec((F, F), lambda r: (0, 0)),
            pl.BlockSpec((1, F), lambda r: (0, 0)),
            pl.BlockSpec((DIM_M, F), lambda r: (0, 0)),
            pl.BlockSpec((DIM_M, F), lambda r: (0, 0)),
            pl.BlockSpec((1, F), lambda r: (0, 0)),
            pl.BlockSpec((1, F), lambda r: (0, 0)),
            pl.BlockSpec((F, F), lambda r: (0, 0)),
            pl.BlockSpec((F, F), lambda r: (0, 0)),
        ],
        out_specs=[
            pl.BlockSpec((ROW_BLK, F), lambda r: (r, 0)),
            pl.BlockSpec((1, 3), lambda r: (0, 0)),
        ],
        out_shape=[
            jax.ShapeDtypeStruct((N, F), jnp.float32),
            jax.ShapeDtypeStruct((1, 3), jnp.float32),
        ],
        scratch_shapes=[
            pltpu.VMEM((N, F), jnp.float32),
            pltpu.SMEM((1,), jnp.float32),
        ],
    )(adj, x, degree, degree, idx2, pe, Wl, bl2, W_gamma, W_beta,
      b_gamma, b_beta, W_add, W_rev)

    inv = np.float32(1.0 / IDX_N)
    l_b = sums[0, 0] * inv
    l_film = (sums[0, 1] + sums[0, 2]) * inv
    return (out, l_b, l_film)


# speedup vs baseline: 1.2347x; 1.0417x over previous
import jax
import jax.numpy as jnp
import numpy as np
from jax.experimental import pallas as pl
from jax.experimental.pallas import tpu as pltpu

DIM_M = 64
D_MAX = 65
OMEGA = 0.1
K_COEF = 1.0
SCALE = DIM_M ** 0.5

N = 10000
F = 128
ROW_BLK = 400
N_ROW = N // ROW_BLK
IDX_N = 1000


def _make_pe():
    pos = np.arange(D_MAX)[:, None].astype(np.float64)
    i = np.arange(DIM_M)[None, :].astype(np.float64)
    pe = pos / np.power(10000.0, (i - i % 2) / DIM_M)
    pe[:, 0::2] = np.sin(pe[:, 0::2])
    pe[:, 1::2] = np.cos(pe[:, 1::2])
    return jnp.asarray(pe, jnp.float32)


def _lrelu(v):
    return jnp.where(v >= 0, v, 0.01 * v)


def _main_kernel(adj_ref, x_ref, deg_ref, degfull_ref, idx_ref, pe_ref,
                 wl_ref, bl_ref, wg_ref, wb_ref, bg_ref, bb_ref,
                 wadd_ref, wrev_ref, out_ref, lb_ref, lf_ref, h_ref, mean_ref,
                 acc_ref):
    r = pl.program_id(0)

    @pl.when(r == 0)
    def _():
        h_ref[...] = SCALE * (jax.lax.dot_general(
            x_ref[...], wl_ref[...], (((1,), (1,)), ((), ())),
            preferred_element_type=jnp.float32) + bl_ref[...])
        mean_ref[0] = (jnp.sum(degfull_ref[...].astype(jnp.float32))
                       / np.float32(N))

    agg = jnp.dot(adj_ref[...], h_ref[...], preferred_element_type=jnp.float32)
    h_row = h_ref[pl.ds(r * ROW_BLK, ROW_BLK), :]

    deg_i = deg_ref[...]
    deg_f = deg_i.astype(jnp.float32)
    is_zero = deg_f == 0.0
    i_feat = jnp.where(is_zero, 0.0, agg / jnp.where(is_zero, 1.0, deg_f))

    oh = (jax.lax.broadcasted_iota(jnp.int32, (ROW_BLK, D_MAX), 1)
          == deg_i).astype(jnp.float32)
    m_dv = jnp.dot(oh, pe_ref[...], preferred_element_type=jnp.float32)
    gamma = _lrelu(jnp.dot(m_dv, wg_ref[...],
                           preferred_element_type=jnp.float32) + bg_ref[...])
    beta = _lrelu(jnp.dot(m_dv, wb_ref[...],
                          preferred_element_type=jnp.float32) + bb_ref[...])

    g1 = gamma + 1.0
    b_add = g1 * jax.lax.dot_general(
        i_feat, wadd_ref[...], (((1,), (1,)), ((), ())),
        preferred_element_type=jnp.float32) + beta
    b_rev = g1 * jax.lax.dot_general(
        i_feat, wrev_ref[...], (((1,), (1,)), ((), ())),
        preferred_element_type=jnp.float32) + beta

    r_mask = (deg_f < mean_ref[0] * K_COEF).astype(jnp.float32)

    bias = OMEGA * (r_mask * b_add - (1.0 - r_mask) * b_rev)
    out_ref[...] = _lrelu((agg + h_row + bias) / (deg_f + 1.0))

    nrm = lambda v: jnp.sqrt(jnp.sum(v * v, axis=1, keepdims=True))
    n_bsel = r_mask * nrm(b_add) + (1.0 - r_mask) * nrm(b_rev)
    row_ids = (r * ROW_BLK
               + jax.lax.broadcasted_iota(jnp.int32, (ROW_BLK, 1), 0))
    cnt = jnp.sum((row_ids == idx_ref[...]).astype(jnp.float32),
                  axis=1, keepdims=True)
    part = jnp.concatenate([
        jnp.sum(cnt * n_bsel).reshape(1, 1),
        jnp.sum(cnt * nrm(gamma)).reshape(1, 1),
        jnp.sum(cnt * nrm(beta)).reshape(1, 1)], axis=1)

    @pl.when(r == 0)
    def _():
        acc_ref[...] = part

    @pl.when(r != 0)
    def _():
        acc_ref[...] += part

    @pl.when(r == N_ROW - 1)
    def _():
        inv = np.float32(1.0 / IDX_N)
        s = acc_ref[...]
        lb_ref[0, 0] = s[0, 0] * inv
        lf_ref[0, 0] = (s[0, 1] + s[0, 2]) * inv


def kernel(x, adj, degree, idx, edge, Wl, bl, W_gamma, W_beta, b_gamma,
           b_beta, W_add, W_rev):
    pe = _make_pe()
    bl2 = bl.reshape(1, F)
    idx2 = idx.reshape(1, IDX_N)

    out, l_b, l_film = pl.pallas_call(
        _main_kernel,
        grid=(N_ROW,),
        in_specs=[
            pl.BlockSpec((ROW_BLK, N), lambda r: (r, 0)),
            pl.BlockSpec((N, F), lambda r: (0, 0)),
            pl.BlockSpec((ROW_BLK, 1), lambda r: (r, 0)),
            pl.BlockSpec((N, 1), lambda r: (0, 0)),
            pl.BlockSpec((1, IDX_N), lambda r: (0, 0)),
            pl.BlockSpec((D_MAX, DIM_M), lambda r: (0, 0)),
            pl.BlockSpec((F, F), lambda r: (0, 0)),
            pl.BlockSpec((1, F), lambda r: (0, 0)),
            pl.BlockSpec((DIM_M, F), lambda r: (0, 0)),
            pl.BlockSpec((DIM_M, F), lambda r: (0, 0)),
            pl.BlockSpec((1, F), lambda r: (0, 0)),
            pl.BlockSpec((1, F), lambda r: (0, 0)),
            pl.BlockSpec((F, F), lambda r: (0, 0)),
            pl.BlockSpec((F, F), lambda r: (0, 0)),
        ],
        out_specs=[
            pl.BlockSpec((ROW_BLK, F), lambda r: (r, 0)),
            pl.BlockSpec(memory_space=pltpu.SMEM),
            pl.BlockSpec(memory_space=pltpu.SMEM),
        ],
        out_shape=[
            jax.ShapeDtypeStruct((N, F), jnp.float32),
            jax.ShapeDtypeStruct((1, 1), jnp.float32),
            jax.ShapeDtypeStruct((1, 1), jnp.float32),
        ],
        scratch_shapes=[
            pltpu.VMEM((N, F), jnp.float32),
            pltpu.SMEM((1,), jnp.float32),
            pltpu.VMEM((1, 3), jnp.float32),
        ],
    )(adj, x, degree, degree, idx2, pe, Wl, bl2, W_gamma, W_beta,
      b_gamma, b_beta, W_add, W_rev)

    return (out, l_b[0, 0], l_film[0, 0])
